# baseline (speedup 1.0000x reference)
import jax
import jax.numpy as jnp
from jax import lax
from jax.experimental import pallas as pl
from jax.experimental.pallas import tpu as pltpu
from jax.experimental.pallas import tpu_sc as plsc
import functools

_N = 1_000_000
_C = 1024

_NW = 32
_SC_S = 131072
_SC_BASE = _N - _SC_S
_EW = _SC_S // _NW
_SC_UNROLL = 8

_TC_FULL = _SC_BASE // _C
_TC_LASTC = _SC_BASE - _C

_LNC = (0.9999999819440732, -0.499999944126168, 0.3333399812235141,
        -0.2500135507449805, 0.19962999455446448, -0.16583248169523926,
        0.14908721977960102, -0.1419603199195033, 0.08632387039390339)
_LN2 = 0.6931471805599453
_SQRT2 = 1.4142135381698608


def _threefry_bits(idx_u32):
    ks0 = jnp.uint32(0)
    ks1 = jnp.uint32(42)
    ks2 = ks0 ^ ks1 ^ jnp.uint32(0x1BD11BDA)
    ks = (ks0, ks1, ks2)
    r0 = (13, 15, 26, 6)
    r1 = (17, 29, 16, 24)

    x0 = jnp.broadcast_to(ks0, idx_u32.shape)
    x1 = idx_u32 + ks1

    def rounds(x0, x1, rots):
        for r in rots:
            x0 = x0 + x1
            x1 = (x1 << jnp.uint32(r)) | (x1 >> jnp.uint32(32 - r))
            x1 = x0 ^ x1
        return x0, x1

    for i, rots in enumerate((r0, r1, r0, r1, r0)):
        x0, x1 = rounds(x0, x1, rots)
        x0 = x0 + ks[(i + 1) % 3]
        x1 = x1 + ks[(i + 2) % 3] + jnp.uint32(i + 1)
    return x0 ^ x1


def _uniform(bits):
    fbits = (bits >> jnp.uint32(9)) | jnp.uint32(0x3F800000)
    f = lax.bitcast_convert_type(fbits, jnp.float32)
    eps = jnp.float32(1e-10)
    return jnp.maximum(eps, (f - jnp.float32(1.0)) + eps)


def _tc_gumbel(gidx):
    u = _uniform(_threefry_bits(gidx.astype(jnp.uint32)))
    return -jnp.log(-jnp.log(u))


def _tc_body(l_ref, idx_ref, val_ref):
    row = lax.broadcasted_iota(jnp.int32, (8, 128), 0)
    col = lax.broadcasted_iota(jnp.int32, (8, 128), 1)
    rc = row * 128 + col

    zm = jnp.full((8, 128), -jnp.inf, jnp.float32)
    im = jnp.zeros((8, 128), jnp.int32)
    bases = [k * _C for k in range(_TC_FULL)] + [_TC_LASTC]
    for base in bases:
        v = jnp.reshape(l_ref[pl.ds(base, _C)], (8, 128))
        z = v + _tc_gumbel(base + rc)
        upd = z > zm
        zm = jnp.where(upd, z, zm)
        im = jnp.where(upd, base + rc, im)

    m = jnp.max(zm)
    cand = jnp.where(zm == m, im, jnp.int32(0x7FFFFFFF))
    idx_ref[0] = jnp.min(cand)
    val_ref[0] = m


def _tc_call(logits):
    return pl.pallas_call(
        _tc_body,
        out_specs=(pl.BlockSpec(memory_space=pltpu.SMEM),
                   pl.BlockSpec(memory_space=pltpu.SMEM)),
        out_shape=(jax.ShapeDtypeStruct((1,), jnp.int32),
                   jax.ShapeDtypeStruct((1,), jnp.float32)),
    )(logits)


def _sc_ln(u):
    ix = lax.bitcast_convert_type(u, jnp.uint32)
    e = (ix >> jnp.uint32(23)).astype(jnp.int32) - 127
    m = lax.bitcast_convert_type(
        (ix & jnp.uint32(0x7FFFFF)) | jnp.uint32(0x3F800000), jnp.float32)
    adj = m > jnp.float32(_SQRT2)
    m = jnp.where(adj, m * jnp.float32(0.5), m)
    ef = jnp.where(adj, e + 1, e).astype(jnp.float32)
    y = m - jnp.float32(1.0)
    p = jnp.full(u.shape, jnp.float32(_LNC[8]))
    for d in range(7, -1, -1):
        p = p * y + jnp.float32(_LNC[d])
    return ef * jnp.float32(_LN2) + y * p


def _sc_body(l_hbm, outv_hbm, outi_hbm, buf, zst, ist):
    wid = lax.axis_index("s") * 2 + lax.axis_index("c")
    base = _SC_BASE + wid * _EW
    pltpu.sync_copy(l_hbm.at[pl.ds(base, _EW)], buf)

    lanes = lax.iota(jnp.int32, 16)

    def step(zm, im, off):
        v = buf[pl.ds(off, 16)]
        u = _uniform(_threefry_bits((base + off + lanes).astype(jnp.uint32)))
        g = -_sc_ln(-_sc_ln(u))
        z = v + g
        upd = z > zm
        return jnp.where(upd, z, zm), jnp.where(upd, base + off + lanes, im)

    def body(it, carry):
        zm, im = carry
        for q in range(_SC_UNROLL):
            zm, im = step(zm, im, it * (16 * _SC_UNROLL) + q * 16)
        return zm, im

    zm0 = jnp.full((16,), -jnp.inf, jnp.float32)
    im0 = jnp.zeros((16,), jnp.int32)
    zm, im = lax.fori_loop(0, _EW // (16 * _SC_UNROLL), body, (zm0, im0))

    zst[...] = zm
    ist[...] = im
    pltpu.sync_copy(zst, outv_hbm.at[wid])
    pltpu.sync_copy(ist, outi_hbm.at[wid])


def _sc_call(logits):
    mesh = plsc.VectorSubcoreMesh(core_axis_name="c", subcore_axis_name="s")
    f = functools.partial(
        pl.kernel,
        out_type=(jax.ShapeDtypeStruct((_NW, 16), jnp.float32),
                  jax.ShapeDtypeStruct((_NW, 16), jnp.int32)),
        mesh=mesh,
        scratch_types=[
            pltpu.VMEM((_EW,), jnp.float32),
            pltpu.VMEM((16,), jnp.float32),
            pltpu.VMEM((16,), jnp.int32),
        ],
    )(_sc_body)
    return f(logits)


def kernel(logits):
    scv, sci = _sc_call(logits)
    tci, tcv = _tc_call(logits)
    vals = jnp.concatenate([scv.reshape(-1), tcv])
    idxs = jnp.concatenate([sci.reshape(-1), tci])
    m = jnp.max(vals)
    cand = jnp.where(vals == m, idxs, jnp.int32(0x7FFFFFFF))
    return jnp.min(cand)


# speedup vs baseline: 2.0239x; 2.0239x over previous
import jax
import jax.numpy as jnp
from jax import lax
from jax.experimental import pallas as pl
from jax.experimental.pallas import tpu as pltpu

_N = 1_000_000
_C = 1024
_NFULL = _N // _C
_LAST = _N - _C

_K1 = 42
_K2 = (42 ^ 0x1BD11BDA) & 0xFFFFFFFF


def _threefry_bits(x1):
    def rot(x, r):
        return (x << jnp.uint32(r)) | (x >> jnp.uint32(32 - r))

    x0 = x1
    x1 = rot(x1, 13) ^ x0
    for r in (15, 26, 6):
        x0 = x0 + x1
        x1 = rot(x1, r) ^ x0
    x0 = x0 + jnp.uint32(_K1)
    x1 = x1 + jnp.uint32((_K2 + 1) & 0xFFFFFFFF)

    for r in (17, 29, 16, 24):
        x0 = x0 + x1
        x1 = rot(x1, r) ^ x0
    x0 = x0 + jnp.uint32(_K2)
    x1 = x1 + jnp.uint32(2)

    for r in (13, 15, 26, 6):
        x0 = x0 + x1
        x1 = rot(x1, r) ^ x0
    x1 = x1 + jnp.uint32((_K1 + 3) & 0xFFFFFFFF)

    for r in (17, 29, 16, 24):
        x0 = x0 + x1
        x1 = rot(x1, r) ^ x0
    x0 = x0 + jnp.uint32(_K1)
    x1 = x1 + jnp.uint32((_K2 + 4) & 0xFFFFFFFF)

    for r in (13, 15, 26, 6):
        x0 = x0 + x1
        x1 = rot(x1, r) ^ x0
    x0 = x0 + jnp.uint32(_K2)
    x1 = x1 + jnp.uint32(5)

    return x0 ^ x1


def _gumbel_from_x1(x1):
    bits = _threefry_bits(x1)
    fbits = (bits >> jnp.uint32(9)) | jnp.uint32(0x3F800000)
    f = lax.bitcast_convert_type(fbits, jnp.float32)
    eps = jnp.float32(1e-10)
    u = (f - jnp.float32(1.0)) + eps
    return -jnp.log(-jnp.log(u))


def _body(l_ref, out_ref):
    row = lax.broadcasted_iota(jnp.int32, (8, 128), 0)
    col = lax.broadcasted_iota(jnp.int32, (8, 128), 1)
    rck = lax.bitcast_convert_type(row * 128 + col + _K1, jnp.uint32)

    zm = jnp.full((8, 128), -jnp.inf, jnp.float32)
    im = jnp.zeros((8, 128), jnp.int32)
    bases = [k * _C for k in range(_NFULL)] + [_LAST]
    for base in bases:
        x1 = rck + jnp.uint32(base)
        z = jnp.reshape(l_ref[pl.ds(base, _C)], (8, 128)) + _gumbel_from_x1(x1)
        upd = z > zm
        zm = jnp.where(upd, z, zm)
        im = jnp.where(upd, lax.bitcast_convert_type(x1, jnp.int32), im)

    m = jnp.max(zm)
    cand = jnp.where(zm == m, im, jnp.int32(0x7FFFFFFF))
    out_ref[0] = jnp.min(cand) - _K1


def kernel(logits):
    out = pl.pallas_call(
        _body,
        out_specs=pl.BlockSpec(memory_space=pltpu.SMEM),
        out_shape=jax.ShapeDtypeStruct((1,), jnp.int32),
    )(logits)
    return out[0]


# speedup vs baseline: 2.1398x; 1.0573x over previous
import jax
import jax.numpy as jnp
from jax import lax
from jax.experimental import pallas as pl
from jax.experimental.pallas import tpu as pltpu

_N = 1_000_000
_C = 1024
_BLK = 262144
_NSTEP = 4

_K1 = 42
_K2 = (42 ^ 0x1BD11BDA) & 0xFFFFFFFF


def _threefry_bits(x1):
    def rot(x, r):
        return (x << jnp.uint32(r)) | (x >> jnp.uint32(32 - r))

    x0 = x1
    x1 = rot(x1, 13) ^ x0
    for r in (15, 26, 6):
        x0 = x0 + x1
        x1 = rot(x1, r) ^ x0
    x0 = x0 + jnp.uint32(_K1)
    x1 = x1 + jnp.uint32((_K2 + 1) & 0xFFFFFFFF)

    for r in (17, 29, 16, 24):
        x0 = x0 + x1
        x1 = rot(x1, r) ^ x0
    x0 = x0 + jnp.uint32(_K2)
    x1 = x1 + jnp.uint32(2)

    for r in (13, 15, 26, 6):
        x0 = x0 + x1
        x1 = rot(x1, r) ^ x0
    x1 = x1 + jnp.uint32((_K1 + 3) & 0xFFFFFFFF)

    for r in (17, 29, 16, 24):
        x0 = x0 + x1
        x1 = rot(x1, r) ^ x0
    x0 = x0 + jnp.uint32(_K1)
    x1 = x1 + jnp.uint32((_K2 + 4) & 0xFFFFFFFF)

    for r in (13, 15, 26, 6):
        x0 = x0 + x1
        x1 = rot(x1, r) ^ x0
    x0 = x0 + jnp.uint32(_K2)
    x1 = x1 + jnp.uint32(5)

    return x0 ^ x1


def _gumbel_from_x1(x1):
    bits = _threefry_bits(x1)
    fbits = (bits >> jnp.uint32(9)) | jnp.uint32(0x3F800000)
    f = lax.bitcast_convert_type(fbits, jnp.float32)
    eps = jnp.float32(1e-10)
    u = (f - jnp.float32(1.0)) + eps
    return -jnp.log(-jnp.log(u))


def _chunk_plan(step):
    gb = step * _BLK
    if step < _NSTEP - 1:
        return [(k * _C, gb + k * _C) for k in range(_BLK // _C)]
    valid = _N - gb
    nfull = valid // _C
    plan = [(k * _C, gb + k * _C) for k in range(nfull)]
    if valid % _C:
        plan.append((valid - _C, _N - _C))
    return plan


def _body(l_ref, out_ref, zms, ims):
    b = pl.program_id(0)
    row = lax.broadcasted_iota(jnp.int32, (8, 128), 0)
    col = lax.broadcasted_iota(jnp.int32, (8, 128), 1)
    rck = lax.bitcast_convert_type(row * 128 + col + _K1, jnp.uint32)

    def accum(zm, im, local, gbase):
        x1 = rck + jnp.uint32(gbase)
        z = jnp.reshape(l_ref[pl.ds(local, _C)], (8, 128)) + _gumbel_from_x1(x1)
        upd = z > zm
        return (jnp.where(upd, z, zm),
                jnp.where(upd, lax.bitcast_convert_type(x1, jnp.int32), im))

    for step in range(_NSTEP):
        @pl.when(b == step)
        def _(step=step):
            if step == 0:
                zm = jnp.full((8, 128), -jnp.inf, jnp.float32)
                im = jnp.zeros((8, 128), jnp.int32)
            else:
                zm = zms[...]
                im = ims[...]
            for local, gbase in _chunk_plan(step):
                zm, im = accum(zm, im, local, gbase)
            if step < _NSTEP - 1:
                zms[...] = zm
                ims[...] = im
            else:
                m = jnp.max(zm)
                cand = jnp.where(zm == m, im, jnp.int32(0x7FFFFFFF))
                out_ref[0] = jnp.min(cand) - _K1


def kernel(logits):
    out = pl.pallas_call(
        _body,
        grid=(_NSTEP,),
        in_specs=[pl.BlockSpec((_BLK,), lambda i: (i,))],
        out_specs=pl.BlockSpec(memory_space=pltpu.SMEM),
        out_shape=jax.ShapeDtypeStruct((1,), jnp.int32),
        scratch_shapes=[
            pltpu.VMEM((8, 128), jnp.float32),
            pltpu.VMEM((8, 128), jnp.int32),
        ],
    )(logits)
    return out[0]
